# baseline (speedup 1.0000x reference)
import functools

import jax
import jax.numpy as jnp
from jax import lax
from jax.experimental import pallas as pl
from jax.experimental.pallas import tpu as pltpu
from jax.experimental.pallas import tpu_sc as plsc

_NSUB = 16
_EB = 125


def _prep_body(x_ref, wt_ref, b_ref, o_ref):
    y = jnp.dot(x_ref[...], wt_ref[...], preferred_element_type=jnp.float32)
    y = y + b_ref[...]
    y0 = y[:, :32]
    y1 = y[:, 32:]
    n0 = jnp.sqrt(jnp.sum(y0 * y0, axis=1, keepdims=True))
    n1 = jnp.sqrt(jnp.sum(y1 * y1, axis=1, keepdims=True))
    o_ref[...] = jnp.concatenate(
        [y0 / jnp.maximum(n0, 1e-12), y1 / jnp.maximum(n1, 1e-12)], axis=1)


def _prep(x, W, b, bn):
    n, K = x.shape
    return pl.pallas_call(
        _prep_body,
        grid=(n // bn,),
        in_specs=[pl.BlockSpec((bn, K), lambda i: (i, 0)),
                  pl.BlockSpec((K, 64), lambda i: (0, 0)),
                  pl.BlockSpec((1, 64), lambda i: (0, 0))],
        out_specs=pl.BlockSpec((bn, 64), lambda i: (i, 0)),
        out_shape=jax.ShapeDtypeStruct((n, 64), jnp.float32),
    )(x, W.T, b.reshape(1, 64))


def _att_body(x1_ref, x2_ref, x3_ref, wt_ref, b1_ref, w2_ref, o_ref):
    xs = [x1_ref[...], x2_ref[...], x3_ref[...]]
    ws = []
    for x in xs:
        h = jnp.dot(x, wt_ref[...], preferred_element_type=jnp.float32)
        h = h + b1_ref[...]
        h = jnp.where(h >= 0, h, 0.2 * h)
        ws.append(jnp.sum(h * w2_ref[...], axis=1, keepdims=True))
    m = jnp.maximum(jnp.maximum(ws[0], ws[1]), ws[2])
    es = [jnp.exp(w - m) for w in ws]
    s = es[0] + es[1] + es[2]
    denom = 1.0 + 3e-6
    o = jnp.zeros_like(xs[0])
    for x, e in zip(xs, es):
        beta = (e / s + 1e-6) / denom
        o = o + beta * x
    o_ref[...] = o


def _att(x1, x2, x3, Wa1, ba1, Wa2, bn):
    n = x1.shape[0]
    spec64 = pl.BlockSpec((bn, 64), lambda i: (i, 0))
    cspec = pl.BlockSpec((64, 64), lambda i: (0, 0))
    rspec = pl.BlockSpec((1, 64), lambda i: (0, 0))
    return pl.pallas_call(
        _att_body,
        grid=(n // bn,),
        in_specs=[spec64, spec64, spec64, cspec, rspec, rspec],
        out_specs=spec64,
        out_shape=jax.ShapeDtypeStruct((n, 64), jnp.float32),
    )(x1, x2, x3, Wa1.T, ba1.reshape(1, 64), Wa2.reshape(1, 64))


def _joint_user_body(u_ref, v_ref, a_ref, wjt_ref, bj_ref,
                     o_ref, colsum_ref, icacc_ref):
    i = pl.program_id(0)
    u = u_ref[...]
    v = v_ref[...]
    s = lax.dot_general(u, v, (((1,), (1,)), ((), ())),
                        preferred_element_type=jnp.float32)
    e = jnp.exp(jnp.where(a_ref[...] > 0, s, -30.0))
    rs = jnp.sum(e, axis=1, keepdims=True)
    uc = jnp.dot(e / rs, v, preferred_element_type=jnp.float32)
    tu = jnp.dot(u, wjt_ref[...], preferred_element_type=jnp.float32)
    tu = jnp.maximum(tu + bj_ref[...], 0.0)
    o_ref[...] = tu + uc
    cs = jnp.sum(e, axis=0)[:, None]
    ica = lax.dot_general(e, u, (((0,), (0,)), ((), ())),
                          preferred_element_type=jnp.float32)

    @pl.when(i == 0)
    def _():
        colsum_ref[...] = cs
        icacc_ref[...] = ica

    @pl.when(i > 0)
    def _():
        colsum_ref[...] = colsum_ref[...] + cs
        icacc_ref[...] = icacc_ref[...] + ica


def _joint_user(u, v, a, Wj, bj, bn):
    n = u.shape[0]
    nv = v.shape[0]
    return pl.pallas_call(
        _joint_user_body,
        grid=(n // bn,),
        in_specs=[pl.BlockSpec((bn, 64), lambda i: (i, 0)),
                  pl.BlockSpec((nv, 64), lambda i: (0, 0)),
                  pl.BlockSpec((bn, nv), lambda i: (i, 0)),
                  pl.BlockSpec((64, 64), lambda i: (0, 0)),
                  pl.BlockSpec((1, 64), lambda i: (0, 0))],
        out_specs=[pl.BlockSpec((bn, 64), lambda i: (i, 0)),
                   pl.BlockSpec((nv, 1), lambda i: (0, 0)),
                   pl.BlockSpec((nv, 64), lambda i: (0, 0))],
        out_shape=[jax.ShapeDtypeStruct((n, 64), jnp.float32),
                   jax.ShapeDtypeStruct((nv, 1), jnp.float32),
                   jax.ShapeDtypeStruct((nv, 64), jnp.float32)],
    )(u, v, a, Wj.T, bj.reshape(1, 64))


def _joint_item_body(u_ref, wjt_ref, bj_ref, uu_ref, colsum_ref, icacc_ref,
                     o_ref):
    u = u_ref[...]
    ti = jnp.dot(u, wjt_ref[...], preferred_element_type=jnp.float32)
    ti = jnp.maximum(ti + bj_ref[...], 0.0)
    ui = ti + icacc_ref[...] / colsum_ref[...]
    o_ref[...] = lax.dot_general(ui, uu_ref[...], (((1,), (1,)), ((), ())),
                                 preferred_element_type=jnp.float32)


def _joint_item_recover(d, Wj, bj, upd_user, colsum, icacc, bn):
    n = d.shape[0]
    nv = upd_user.shape[0]
    return pl.pallas_call(
        _joint_item_body,
        grid=(n // bn,),
        in_specs=[pl.BlockSpec((bn, 64), lambda i: (i, 0)),
                  pl.BlockSpec((64, 64), lambda i: (0, 0)),
                  pl.BlockSpec((1, 64), lambda i: (0, 0)),
                  pl.BlockSpec((nv, 64), lambda i: (0, 0)),
                  pl.BlockSpec((bn, 1), lambda i: (i, 0)),
                  pl.BlockSpec((bn, 64), lambda i: (i, 0))],
        out_specs=pl.BlockSpec((bn, nv), lambda i: (i, 0)),
        out_shape=jax.ShapeDtypeStruct((n, nv), jnp.float32),
    )(d, Wj.T, bj.reshape(1, 64), upd_user, colsum, icacc)


def _splat_sum16(v):
    idx = lax.iota(jnp.int32, 16)
    for sh in (8, 4, 2, 1):
        v = v + v.at[jnp.bitwise_xor(idx, sh)].get(mode="promise_in_bounds")
    return v


_INV_SQRT2 = 0.7071067811865476


def _rsqrt16(x):
    y = jnp.full((16,), (2.0 ** 40) * _INV_SQRT2, jnp.float32)
    for k in range(-39, 18):
        y = jnp.where(x >= 4.0 ** k, jnp.float32((2.0 ** -k) * _INV_SQRT2), y)
    for _ in range(6):
        y = y * (1.5 - 0.5 * x * y * y)
    return y


def _graph_params(n, m):
    mt = m // _NSUB
    assert mt % _EB == 0
    nb = mt // _EB
    R = ((n + _NSUB - 1) // _NSUB + 7) // 8 * 8
    npd = R * _NSUB
    return nb, npd, R


def _make_pair_routing(nA, mA, nB, mB):
    nbA, npA, RA = _graph_params(nA, mA)
    nbB, npB, RB = _graph_params(nB, mB)
    Rmax = max(RA, RB)
    npmax = max(npA, npB)
    nbmax = max(nbA, nbB)

    mesh = plsc.VectorSubcoreMesh(core_axis_name="c", subcore_axis_name="s")

    @functools.partial(
        pl.kernel,
        out_type=[jax.ShapeDtypeStruct((npA, 64), jnp.float32),
                  jax.ShapeDtypeStruct((npB, 64), jnp.float32)],
        mesh=mesh,
        scratch_types=[
            pltpu.VMEM((nbmax, _EB), jnp.int32),
            pltpu.VMEM((nbmax, _EB), jnp.int32),
            pltpu.VMEM((_EB, 64), jnp.float32),
            pltpu.VMEM((_EB, 64), jnp.float32),
            pltpu.VMEM((_EB, 64), jnp.float32),
            pltpu.VMEM((_EB, 64), jnp.float32),
            pltpu.VMEM((_EB, 64), jnp.float32),
            pltpu.VMEM((_EB, 64), jnp.float32),
            pltpu.VMEM((Rmax, 64), jnp.float32),
            pltpu.VMEM((Rmax, 64), jnp.float32),
            pltpu.VMEM_SHARED((npmax, 64), jnp.float32),
            pltpu.SemaphoreType.DMA,
            pltpu.SemaphoreType.DMA,
            pltpu.SemaphoreType.DMA,
            pltpu.SemaphoreType.DMA,
        ],
        compiler_params=pltpu.CompilerParams(use_tc_tiling_on_sc=False),
    )
    def routing(cA0, srcA3, trgA3, cB0, srcB3, trgB3,
                cA, cB, sv, tv,
                zb0, zb1, cb0, cb1, wb0, wb1, rb, ab,
                acc, gsem0, gsem1, ssem0, ssem1):
        cid = lax.axis_index("c")
        sid = lax.axis_index("s")
        zbs, cbs, wbs = (zb0, zb1), (cb0, cb1), (wb0, wb1)
        gsems, ssems = (gsem0, gsem1), (ssem0, ssem1)
        iota16 = lax.iota(jnp.int32, 16)

        def zero_rows(buf, R):
            def zrow(r, _):
                zv = jnp.zeros((16,), jnp.float32)
                buf[r, 0:16] = zv
                buf[r, 16:32] = zv
                buf[r, 32:48] = zv
                buf[r, 48:64] = zv
                return 0
            lax.fori_loop(0, R, zrow, 0)

        def run(c0_hbm, src3, trg3, c_hbm, nb, R):
            base = sid * R
            rows = pl.ds(base, R)
            lrows = pl.ds(0, R)
            pltpu.sync_copy(src3.at[sid], sv.at[pl.ds(0, nb)])
            pltpu.sync_copy(trg3.at[sid], tv.at[pl.ds(0, nb)])
            pltpu.sync_copy(c0_hbm.at[rows], rb.at[lrows])
            pltpu.sync_copy(rb.at[lrows], c_hbm.at[rows])
            zero_rows(ab, R)
            pltpu.sync_copy(ab.at[lrows], acc.at[rows])
            plsc.subcore_barrier()

            def issue_gathers(b, k):
                pltpu.async_copy(c0_hbm.at[sv.at[b]], zbs[k], gsems[k])
                pltpu.async_copy(c_hbm.at[tv.at[b]], cbs[k], gsems[k])

            def wait_gathers(b, k):
                pltpu.make_async_copy(c0_hbm.at[sv.at[b]], zbs[k], gsems[k]).wait()
                pltpu.make_async_copy(c_hbm.at[tv.at[b]], cbs[k], gsems[k]).wait()

            def one_iter(it, _):
                issue_gathers(0, 0)
                issue_gathers(1, 1)

                def pairstep(j, _):
                    for k in (0, 1):
                        b = 2 * j + k
                        wait_gathers(b, k)

                        @pl.when(b >= 2)
                        def _():
                            pltpu.make_async_copy(
                                wbs[k], acc.at[tv.at[b]], ssems[k]).wait()

                        zb, cb, wb = zbs[k], cbs[k], wbs[k]

                        def edge(e, _):
                            z0 = zb[e, 0:16]
                            z1 = zb[e, 16:32]
                            z2 = zb[e, 32:48]
                            z3 = zb[e, 48:64]
                            d0 = cb[e, 0:16]
                            d1 = cb[e, 16:32]
                            d2 = cb[e, 32:48]
                            d3 = cb[e, 48:64]
                            t = z0 * d0 + z1 * d1 - z2 * d2 - z3 * d3
                            dv = _splat_sum16(t)
                            w0 = 1.0 / (1.0 + jnp.exp(-dv))
                            w1 = 1.0 - w0
                            wb[e, 0:16] = w0 * z0
                            wb[e, 16:32] = w0 * z1
                            wb[e, 32:48] = w1 * z2
                            wb[e, 48:64] = w1 * z3
                            return 0

                        lax.fori_loop(0, _EB, edge, 0, unroll=2)

                        pltpu.async_copy(wb, acc.at[tv.at[b]], ssems[k],
                                         add=True)

                        @pl.when(b + 2 < nb)
                        def _():
                            issue_gathers(b + 2, k)
                    return 0

                lax.fori_loop(0, nb // 2, pairstep, 0)
                for k in (0, 1):
                    pltpu.make_async_copy(wbs[k], acc.at[tv.at[0]],
                                          ssems[k]).wait()
                plsc.subcore_barrier()

                pltpu.sync_copy(c_hbm.at[rows], rb.at[lrows])
                pltpu.sync_copy(acc.at[rows], ab.at[lrows])

                def ngrp(g, _):
                    r0 = g * 16
                    z16 = jnp.zeros((16,), jnp.float32)

                    def prow(tt, carry):
                        c0v, c1v = carry
                        r = r0 + tt
                        a0 = rb[r, 0:16] + ab[r, 0:16]
                        a1 = rb[r, 16:32] + ab[r, 16:32]
                        a2 = rb[r, 32:48] + ab[r, 32:48]
                        a3 = rb[r, 48:64] + ab[r, 48:64]
                        rb[r, 0:16] = a0
                        rb[r, 16:32] = a1
                        rb[r, 32:48] = a2
                        rb[r, 48:64] = a3
                        ss0 = _splat_sum16(a0 * a0 + a1 * a1)
                        ss1 = _splat_sum16(a2 * a2 + a3 * a3)
                        sel = iota16 == tt
                        return (jnp.where(sel, ss0, c0v),
                                jnp.where(sel, ss1, c1v))

                    d0, d1 = lax.fori_loop(0, 16, prow, (z16, z16))
                    iv0 = _rsqrt16(jnp.maximum(d0, 1e-24))
                    iv1 = _rsqrt16(jnp.maximum(d1, 1e-24))

                    def srow(tt, _):
                        r = r0 + tt
                        tvec = jnp.zeros((16,), jnp.int32) + tt
                        s0 = iv0.at[tvec].get(mode="promise_in_bounds")
                        s1 = iv1.at[tvec].get(mode="promise_in_bounds")
                        rb[r, 0:16] = rb[r, 0:16] * s0
                        rb[r, 16:32] = rb[r, 16:32] * s0
                        rb[r, 32:48] = rb[r, 32:48] * s1
                        rb[r, 48:64] = rb[r, 48:64] * s1
                        ab[r, 0:16] = z16
                        ab[r, 16:32] = z16
                        ab[r, 32:48] = z16
                        ab[r, 48:64] = z16
                        return 0

                    lax.fori_loop(0, 16, srow, 0)
                    return 0

                lax.fori_loop(0, R // 16, ngrp, 0)
                pltpu.sync_copy(rb.at[lrows], c_hbm.at[rows])
                pltpu.sync_copy(ab.at[lrows], acc.at[rows])
                plsc.subcore_barrier()
                return 0

            lax.fori_loop(0, 3, one_iter, 0)

        @pl.when(cid == 0)
        def _():
            run(cA0, srcA3, trgA3, cA, nbA, RA)

        @pl.when(cid == 1)
        def _():
            run(cB0, srcB3, trgB3, cB, nbB, RB)

    return routing


def _route_inputs(c0, edge_index, n, m):
    nb, npd, _ = _graph_params(n, m)
    c0p = jnp.pad(c0, ((0, npd - n), (0, 0)))
    src = edge_index[0].astype(jnp.int32).reshape(_NSUB, nb, _EB)
    trg = edge_index[1].astype(jnp.int32).reshape(_NSUB, nb, _EB)
    return c0p, src, trg


def kernel(NC_1, NC_2, NC_3, D_1, D_2, D_3, edge_indexNC1, edge_indexNC2,
           edge_indexNC3, edge_indexD1, edge_indexD2, edge_indexD3,
           heterogeneous, heterogeneous1, A, L, W_nc, b_nc, W_d, b_d,
           Wa1, ba1, Wa2, Wj, bj):
    n_nc, n_d = NC_1.shape[0], D_1.shape[0]
    m_nc, m_d = edge_indexNC1.shape[1], edge_indexD1.shape[1]

    x_nc1 = _prep(NC_1, W_nc, b_nc, 200)
    x_nc2 = _prep(NC_2, W_nc, b_nc, 200)
    x_nc3 = _prep(NC_3, W_nc, b_nc, 200)
    x_d1 = _prep(D_1, W_d, b_d, 200)
    x_d2 = _prep(D_2, W_d, b_d, 200)
    x_d3 = _prep(D_3, W_d, b_d, 200)

    g_nc1 = _route_inputs(x_nc1, edge_indexNC1, n_nc, m_nc)
    g_nc2 = _route_inputs(x_nc2, edge_indexNC2, n_nc, m_nc)
    g_nc3 = _route_inputs(x_nc3, edge_indexNC3, n_nc, m_nc)
    g_d1 = _route_inputs(x_d1, edge_indexD1, n_d, m_d)
    g_d2 = _route_inputs(x_d2, edge_indexD2, n_d, m_d)
    g_d3 = _route_inputs(x_d3, edge_indexD3, n_d, m_d)

    route_nn = _make_pair_routing(n_nc, m_nc, n_nc, m_nc)
    route_nd = _make_pair_routing(n_nc, m_nc, n_d, m_d)
    route_dd = _make_pair_routing(n_d, m_d, n_d, m_d)

    nc1p, nc2p = route_nn(*g_nc1, *g_nc2)
    nc3p, d1p = route_nd(*g_nc3, *g_d1)
    d2p, d3p = route_dd(*g_d2, *g_d3)

    nc1, nc2, nc3 = nc1p[:n_nc], nc2p[:n_nc], nc3p[:n_nc]
    d1, d2, d3 = d1p[:n_d], d2p[:n_d], d3p[:n_d]

    nc = _att(nc1, nc2, nc3, Wa1, ba1, Wa2, 200)
    d = _att(d1, d2, d3, Wa1, ba1, Wa2, 200)

    upd_user, colsum, icacc = _joint_user(nc, d, A, Wj, bj, 200)
    recover = _joint_item_recover(d, Wj, bj, upd_user, colsum, icacc, 200)
    return recover


# speedup vs baseline: 3.1696x; 3.1696x over previous
import functools

import jax
import jax.numpy as jnp
from jax import lax
from jax.experimental import pallas as pl
from jax.experimental.pallas import tpu as pltpu
from jax.experimental.pallas import tpu_sc as plsc

_NSUB = 16
_EB = 125


def _prep_body(x_ref, wt_ref, b_ref, o_ref):
    y = jnp.dot(x_ref[...], wt_ref[...], preferred_element_type=jnp.float32)
    y = y + b_ref[...]
    y0 = y[:, :32]
    y1 = y[:, 32:]
    n0 = jnp.sqrt(jnp.sum(y0 * y0, axis=1, keepdims=True))
    n1 = jnp.sqrt(jnp.sum(y1 * y1, axis=1, keepdims=True))
    o_ref[...] = jnp.concatenate(
        [y0 / jnp.maximum(n0, 1e-12), y1 / jnp.maximum(n1, 1e-12)], axis=1)


def _prep(x, W, b, bn):
    n, K = x.shape
    return pl.pallas_call(
        _prep_body,
        grid=(n // bn,),
        in_specs=[pl.BlockSpec((bn, K), lambda i: (i, 0)),
                  pl.BlockSpec((K, 64), lambda i: (0, 0)),
                  pl.BlockSpec((1, 64), lambda i: (0, 0))],
        out_specs=pl.BlockSpec((bn, 64), lambda i: (i, 0)),
        out_shape=jax.ShapeDtypeStruct((n, 64), jnp.float32),
    )(x, W.T, b.reshape(1, 64))


def _att_body(x1_ref, x2_ref, x3_ref, wt_ref, b1_ref, w2_ref, o_ref):
    xs = [x1_ref[...], x2_ref[...], x3_ref[...]]
    ws = []
    for x in xs:
        h = jnp.dot(x, wt_ref[...], preferred_element_type=jnp.float32)
        h = h + b1_ref[...]
        h = jnp.where(h >= 0, h, 0.2 * h)
        ws.append(jnp.sum(h * w2_ref[...], axis=1, keepdims=True))
    m = jnp.maximum(jnp.maximum(ws[0], ws[1]), ws[2])
    es = [jnp.exp(w - m) for w in ws]
    s = es[0] + es[1] + es[2]
    denom = 1.0 + 3e-6
    o = jnp.zeros_like(xs[0])
    for x, e in zip(xs, es):
        beta = (e / s + 1e-6) / denom
        o = o + beta * x
    o_ref[...] = o


def _att(x1, x2, x3, Wa1, ba1, Wa2, bn):
    n = x1.shape[0]
    spec64 = pl.BlockSpec((bn, 64), lambda i: (i, 0))
    cspec = pl.BlockSpec((64, 64), lambda i: (0, 0))
    rspec = pl.BlockSpec((1, 64), lambda i: (0, 0))
    return pl.pallas_call(
        _att_body,
        grid=(n // bn,),
        in_specs=[spec64, spec64, spec64, cspec, rspec, rspec],
        out_specs=spec64,
        out_shape=jax.ShapeDtypeStruct((n, 64), jnp.float32),
    )(x1, x2, x3, Wa1.T, ba1.reshape(1, 64), Wa2.reshape(1, 64))


def _joint_user_body(u_ref, v_ref, a_ref, wjt_ref, bj_ref,
                     o_ref, colsum_ref, icacc_ref):
    i = pl.program_id(0)
    u = u_ref[...]
    v = v_ref[...]
    s = lax.dot_general(u, v, (((1,), (1,)), ((), ())),
                        preferred_element_type=jnp.float32)
    e = jnp.exp(jnp.where(a_ref[...] > 0, s, -30.0))
    rs = jnp.sum(e, axis=1, keepdims=True)
    uc = jnp.dot(e / rs, v, preferred_element_type=jnp.float32)
    tu = jnp.dot(u, wjt_ref[...], preferred_element_type=jnp.float32)
    tu = jnp.maximum(tu + bj_ref[...], 0.0)
    o_ref[...] = tu + uc
    cs = jnp.sum(e, axis=0)[:, None]
    ica = lax.dot_general(e, u, (((0,), (0,)), ((), ())),
                          preferred_element_type=jnp.float32)

    @pl.when(i == 0)
    def _():
        colsum_ref[...] = cs
        icacc_ref[...] = ica

    @pl.when(i > 0)
    def _():
        colsum_ref[...] = colsum_ref[...] + cs
        icacc_ref[...] = icacc_ref[...] + ica


def _joint_user(u, v, a, Wj, bj, bn):
    n = u.shape[0]
    nv = v.shape[0]
    return pl.pallas_call(
        _joint_user_body,
        grid=(n // bn,),
        in_specs=[pl.BlockSpec((bn, 64), lambda i: (i, 0)),
                  pl.BlockSpec((nv, 64), lambda i: (0, 0)),
                  pl.BlockSpec((bn, nv), lambda i: (i, 0)),
                  pl.BlockSpec((64, 64), lambda i: (0, 0)),
                  pl.BlockSpec((1, 64), lambda i: (0, 0))],
        out_specs=[pl.BlockSpec((bn, 64), lambda i: (i, 0)),
                   pl.BlockSpec((nv, 1), lambda i: (0, 0)),
                   pl.BlockSpec((nv, 64), lambda i: (0, 0))],
        out_shape=[jax.ShapeDtypeStruct((n, 64), jnp.float32),
                   jax.ShapeDtypeStruct((nv, 1), jnp.float32),
                   jax.ShapeDtypeStruct((nv, 64), jnp.float32)],
    )(u, v, a, Wj.T, bj.reshape(1, 64))


def _joint_item_body(u_ref, wjt_ref, bj_ref, uu_ref, colsum_ref, icacc_ref,
                     o_ref):
    u = u_ref[...]
    ti = jnp.dot(u, wjt_ref[...], preferred_element_type=jnp.float32)
    ti = jnp.maximum(ti + bj_ref[...], 0.0)
    ui = ti + icacc_ref[...] / colsum_ref[...]
    o_ref[...] = lax.dot_general(ui, uu_ref[...], (((1,), (1,)), ((), ())),
                                 preferred_element_type=jnp.float32)


def _joint_item_recover(d, Wj, bj, upd_user, colsum, icacc, bn):
    n = d.shape[0]
    nv = upd_user.shape[0]
    return pl.pallas_call(
        _joint_item_body,
        grid=(n // bn,),
        in_specs=[pl.BlockSpec((bn, 64), lambda i: (i, 0)),
                  pl.BlockSpec((64, 64), lambda i: (0, 0)),
                  pl.BlockSpec((1, 64), lambda i: (0, 0)),
                  pl.BlockSpec((nv, 64), lambda i: (0, 0)),
                  pl.BlockSpec((bn, 1), lambda i: (i, 0)),
                  pl.BlockSpec((bn, 64), lambda i: (i, 0))],
        out_specs=pl.BlockSpec((bn, nv), lambda i: (i, 0)),
        out_shape=jax.ShapeDtypeStruct((n, nv), jnp.float32),
    )(d, Wj.T, bj.reshape(1, 64), upd_user, colsum, icacc)


def _splat_sum16(v):
    idx = lax.iota(jnp.int32, 16)
    for sh in (8, 4, 2, 1):
        v = v + v.at[jnp.bitwise_xor(idx, sh)].get(mode="promise_in_bounds")
    return v


_INV_SQRT2 = 0.7071067811865476


def _rsqrt16(x):
    y = jnp.full((16,), (2.0 ** 40) * _INV_SQRT2, jnp.float32)
    for k in range(-39, 18):
        y = jnp.where(x >= 4.0 ** k, jnp.float32((2.0 ** -k) * _INV_SQRT2), y)
    for _ in range(6):
        y = y * (1.5 - 0.5 * x * y * y)
    return y


def _graph_params(n, m):
    mt = m // _NSUB
    assert mt % _EB == 0
    nb = mt // _EB
    R = ((n + _NSUB - 1) // _NSUB + 7) // 8 * 8
    npd = R * _NSUB
    return nb, npd, R


def _make_pair_routing(nA, mA, nB, mB):
    nbA, npA, RA = _graph_params(nA, mA)
    nbB, npB, RB = _graph_params(nB, mB)
    Rmax = max(RA, RB)
    npmax = max(npA, npB)
    nbmax = max(nbA, nbB)

    mesh = plsc.VectorSubcoreMesh(core_axis_name="c", subcore_axis_name="s")

    @functools.partial(
        pl.kernel,
        out_type=[jax.ShapeDtypeStruct((npA, 64), jnp.float32),
                  jax.ShapeDtypeStruct((npB, 64), jnp.float32)],
        mesh=mesh,
        scratch_types=[
            pltpu.VMEM((nbmax, _EB), jnp.int32),
            pltpu.VMEM((nbmax, _EB), jnp.int32),
            pltpu.VMEM((_EB, 64), jnp.float32),
            pltpu.VMEM((_EB, 64), jnp.float32),
            pltpu.VMEM((_EB, 64), jnp.float32),
            pltpu.VMEM((_EB, 64), jnp.float32),
            pltpu.VMEM((_EB, 64), jnp.float32),
            pltpu.VMEM((_EB, 64), jnp.float32),
            pltpu.VMEM((Rmax, 64), jnp.float32),
            pltpu.VMEM((Rmax, 64), jnp.float32),
            pltpu.VMEM_SHARED((npmax, 64), jnp.float32),
            pltpu.SemaphoreType.DMA,
            pltpu.SemaphoreType.DMA,
            pltpu.SemaphoreType.DMA,
            pltpu.SemaphoreType.DMA,
        ],
        compiler_params=pltpu.CompilerParams(use_tc_tiling_on_sc=False),
    )
    def routing(cA0, srcA3, trgA3, cB0, srcB3, trgB3,
                cA, cB, sv, tv,
                zb0, zb1, cb0, cb1, wb0, wb1, rb, ab,
                acc, gsem0, gsem1, ssem0, ssem1):
        cid = lax.axis_index("c")
        sid = lax.axis_index("s")
        zbs, cbs, wbs = (zb0, zb1), (cb0, cb1), (wb0, wb1)
        gsems, ssems = (gsem0, gsem1), (ssem0, ssem1)
        iota16 = lax.iota(jnp.int32, 16)

        def zero_rows(buf, R):
            def zrow(r, _):
                zv = jnp.zeros((16,), jnp.float32)
                buf[r, 0:16] = zv
                buf[r, 16:32] = zv
                buf[r, 32:48] = zv
                buf[r, 48:64] = zv
                return 0
            lax.fori_loop(0, R, zrow, 0)

        def run(c0_hbm, src3, trg3, c_hbm, nb, R):
            base = sid * R
            rows = pl.ds(base, R)
            lrows = pl.ds(0, R)
            pltpu.sync_copy(src3.at[sid], sv.at[pl.ds(0, nb)])
            pltpu.sync_copy(trg3.at[sid], tv.at[pl.ds(0, nb)])
            pltpu.sync_copy(c0_hbm.at[rows], rb.at[lrows])
            pltpu.sync_copy(rb.at[lrows], c_hbm.at[rows])
            zero_rows(ab, R)
            pltpu.sync_copy(ab.at[lrows], acc.at[rows])
            plsc.subcore_barrier()

            def issue_gathers(b, k):
                pltpu.async_copy(c0_hbm.at[sv.at[b]], zbs[k], gsems[k])
                pltpu.async_copy(c_hbm.at[tv.at[b]], cbs[k], gsems[k])

            def wait_gathers(b, k):
                pltpu.make_async_copy(c0_hbm.at[sv.at[b]], zbs[k], gsems[k]).wait()
                pltpu.make_async_copy(c_hbm.at[tv.at[b]], cbs[k], gsems[k]).wait()

            def one_iter(it, _):
                issue_gathers(0, 0)
                issue_gathers(1, 1)

                def pairstep(j, _):
                    for k in (0, 1):
                        b = 2 * j + k
                        wait_gathers(b, k)

                        @pl.when(b >= 2)
                        def _():
                            pltpu.make_async_copy(
                                wbs[k], acc.at[tv.at[b]], ssems[k]).wait()

                        zb, cb, wb = zbs[k], cbs[k], wbs[k]

                        def edge(e, _):
                            z0 = zb[e, 0:16]
                            z1 = zb[e, 16:32]
                            z2 = zb[e, 32:48]
                            z3 = zb[e, 48:64]
                            d0 = cb[e, 0:16]
                            d1 = cb[e, 16:32]
                            d2 = cb[e, 32:48]
                            d3 = cb[e, 48:64]
                            t = z0 * d0 + z1 * d1 - z2 * d2 - z3 * d3
                            dv = _splat_sum16(t)
                            w0 = 1.0 / (1.0 + jnp.exp(-dv))
                            w1 = 1.0 - w0
                            wb[e, 0:16] = w0 * z0
                            wb[e, 16:32] = w0 * z1
                            wb[e, 32:48] = w1 * z2
                            wb[e, 48:64] = w1 * z3
                            return 0

                        lax.fori_loop(0, _EB, edge, 0)

                        pltpu.async_copy(wb, acc.at[tv.at[b]], ssems[k],
                                         add=True)

                        @pl.when(b + 2 < nb)
                        def _():
                            issue_gathers(b + 2, k)
                    return 0

                lax.fori_loop(0, nb // 2, pairstep, 0)
                for k in (0, 1):
                    pltpu.make_async_copy(wbs[k], acc.at[tv.at[0]],
                                          ssems[k]).wait()
                plsc.subcore_barrier()

                pltpu.sync_copy(c_hbm.at[rows], rb.at[lrows])
                pltpu.sync_copy(acc.at[rows], ab.at[lrows])

                def ngrp(g, _):
                    r0 = g * 16
                    z16 = jnp.zeros((16,), jnp.float32)

                    def prow(tt, carry):
                        c0v, c1v = carry
                        r = r0 + tt
                        a0 = rb[r, 0:16] + ab[r, 0:16]
                        a1 = rb[r, 16:32] + ab[r, 16:32]
                        a2 = rb[r, 32:48] + ab[r, 32:48]
                        a3 = rb[r, 48:64] + ab[r, 48:64]
                        rb[r, 0:16] = a0
                        rb[r, 16:32] = a1
                        rb[r, 32:48] = a2
                        rb[r, 48:64] = a3
                        ss0 = _splat_sum16(a0 * a0 + a1 * a1)
                        ss1 = _splat_sum16(a2 * a2 + a3 * a3)
                        sel = iota16 == tt
                        return (jnp.where(sel, ss0, c0v),
                                jnp.where(sel, ss1, c1v))

                    d0, d1 = lax.fori_loop(0, 16, prow, (z16, z16))
                    iv0 = _rsqrt16(jnp.maximum(d0, 1e-24))
                    iv1 = _rsqrt16(jnp.maximum(d1, 1e-24))

                    def srow(tt, _):
                        r = r0 + tt
                        tvec = jnp.zeros((16,), jnp.int32) + tt
                        s0 = iv0.at[tvec].get(mode="promise_in_bounds")
                        s1 = iv1.at[tvec].get(mode="promise_in_bounds")
                        rb[r, 0:16] = rb[r, 0:16] * s0
                        rb[r, 16:32] = rb[r, 16:32] * s0
                        rb[r, 32:48] = rb[r, 32:48] * s1
                        rb[r, 48:64] = rb[r, 48:64] * s1
                        ab[r, 0:16] = z16
                        ab[r, 16:32] = z16
                        ab[r, 32:48] = z16
                        ab[r, 48:64] = z16
                        return 0

                    lax.fori_loop(0, 16, srow, 0)
                    return 0

                lax.fori_loop(0, R // 16, ngrp, 0)
                pltpu.sync_copy(rb.at[lrows], c_hbm.at[rows])
                pltpu.sync_copy(ab.at[lrows], acc.at[rows])
                plsc.subcore_barrier()
                return 0

            lax.fori_loop(0, 3, one_iter, 0)

        @pl.when(cid == 0)
        def _():
            run(cA0, srcA3, trgA3, cA, nbA, RA)

        @pl.when(cid == 1)
        def _():
            run(cB0, srcB3, trgB3, cB, nbB, RB)

    return routing


def _route_inputs(c0, edge_index, n, m):
    nb, npd, _ = _graph_params(n, m)
    c0p = jnp.pad(c0, ((0, npd - n), (0, 0)))
    src = edge_index[0].astype(jnp.int32).reshape(_NSUB, nb, _EB)
    trg = edge_index[1].astype(jnp.int32).reshape(_NSUB, nb, _EB)
    return c0p, src, trg


def kernel(NC_1, NC_2, NC_3, D_1, D_2, D_3, edge_indexNC1, edge_indexNC2,
           edge_indexNC3, edge_indexD1, edge_indexD2, edge_indexD3,
           heterogeneous, heterogeneous1, A, L, W_nc, b_nc, W_d, b_d,
           Wa1, ba1, Wa2, Wj, bj):
    n_nc, n_d = NC_1.shape[0], D_1.shape[0]
    m_nc, m_d = edge_indexNC1.shape[1], edge_indexD1.shape[1]

    x_nc1 = _prep(NC_1, W_nc, b_nc, 200)
    x_nc2 = _prep(NC_2, W_nc, b_nc, 200)
    x_nc3 = _prep(NC_3, W_nc, b_nc, 200)
    x_d1 = _prep(D_1, W_d, b_d, 200)
    x_d2 = _prep(D_2, W_d, b_d, 200)
    x_d3 = _prep(D_3, W_d, b_d, 200)

    g_nc1 = _route_inputs(x_nc1, edge_indexNC1, n_nc, m_nc)
    g_nc2 = _route_inputs(x_nc2, edge_indexNC2, n_nc, m_nc)
    g_nc3 = _route_inputs(x_nc3, edge_indexNC3, n_nc, m_nc)
    g_d1 = _route_inputs(x_d1, edge_indexD1, n_d, m_d)
    g_d2 = _route_inputs(x_d2, edge_indexD2, n_d, m_d)
    g_d3 = _route_inputs(x_d3, edge_indexD3, n_d, m_d)

    route_nn = _make_pair_routing(n_nc, m_nc, n_nc, m_nc)
    route_nd = _make_pair_routing(n_nc, m_nc, n_d, m_d)
    route_dd = _make_pair_routing(n_d, m_d, n_d, m_d)

    nc1p, nc2p = route_nn(*g_nc1, *g_nc2)
    nc3p, d1p = route_nd(*g_nc3, *g_d1)
    d2p, d3p = route_dd(*g_d2, *g_d3)

    nc1, nc2, nc3 = nc1p[:n_nc], nc2p[:n_nc], nc3p[:n_nc]
    d1, d2, d3 = d1p[:n_d], d2p[:n_d], d3p[:n_d]

    nc = _att(nc1, nc2, nc3, Wa1, ba1, Wa2, 200)
    d = _att(d1, d2, d3, Wa1, ba1, Wa2, 200)

    upd_user, colsum, icacc = _joint_user(nc, d, A, Wj, bj, 200)
    recover = _joint_item_recover(d, Wj, bj, upd_user, colsum, icacc, 200)
    return recover
